# baseline (speedup 1.0000x reference)
import functools

import numpy as np
import jax
import jax.numpy as jnp
from jax import lax
from jax.experimental import pallas as pl
from jax.experimental.pallas import tpu as pltpu
from jax.experimental.pallas import tpu_sc as plsc

_N = 10000
_E = 320000
_D = 128
_G = 64
_NCLS = 2
_NEG = 0.2
_EPS_BN = 1e-5
_BLK = 1000

_NCORE = 2
_NSUB = 16
_NW = _NCORE * _NSUB
_EPW = _E // _NW
_CB = 80
_NCHUNK = _EPW // _CB
_NG = _CB // 16
_NPAD = 10240
_RPT = _NPAD // _NSUB


def _mm_body(x_ref, wl_ref, bl_ref, wr_ref, br_ref, xl_ref, xr_ref):
    x = x_ref[...]
    xl_ref[...] = jnp.dot(x, wl_ref[...],
                          preferred_element_type=jnp.float32) + bl_ref[...]
    xr_ref[...] = jnp.dot(x, wr_ref[...],
                          preferred_element_type=jnp.float32) + br_ref[...]


def _mm(x, Wl, bl, Wr, br_):
    return pl.pallas_call(
        _mm_body,
        grid=(_N // _BLK,),
        in_specs=[
            pl.BlockSpec((_BLK, _D), lambda i: (i, 0)),
            pl.BlockSpec((_D, _D), lambda i: (0, 0)),
            pl.BlockSpec((_D,), lambda i: (0,)),
            pl.BlockSpec((_D, _D), lambda i: (0, 0)),
            pl.BlockSpec((_D,), lambda i: (0,)),
        ],
        out_specs=[
            pl.BlockSpec((_BLK, _D), lambda i: (i, 0)),
            pl.BlockSpec((_BLK, _D), lambda i: (i, 0)),
        ],
        out_shape=[
            jax.ShapeDtypeStruct((_N, _D), jnp.float32),
            jax.ShapeDtypeStruct((_N, _D), jnp.float32),
        ],
    )(x, Wl, bl, Wr, br_)


def _edge_body(xl_hbm, xr_hbm, src_hbm, dst_hbm, ea_hbm, attb_hbm, web_hbm,
               z2d_hbm, z1d_hbm, num_out, den_out,
               src_v, dst_v, ds_v, ea_v, xl_v, xr_v, w_v, att_v, we_v, den_v,
               num_sh, sem, ssem):
    cid = lax.axis_index("c")
    sid = lax.axis_index("s")
    wid = cid * _NSUB + sid

    pltpu.sync_copy(attb_hbm, att_v)
    pltpu.sync_copy(web_hbm, we_v)
    pltpu.sync_copy(z1d_hbm, den_v)
    pltpu.sync_copy(z2d_hbm, num_sh.at[pl.ds(sid * _RPT, _RPT)])
    plsc.subcore_barrier()

    iota16 = lax.iota(jnp.int32, 16)
    rowvs = [iota16 + 16 * g for g in range(_NG)]

    def chunk_body(j, carry):
        start = pl.multiple_of(wid * _EPW + j * _CB, 8)
        pltpu.sync_copy(src_hbm.at[pl.ds(start, _CB)], src_v)
        pltpu.sync_copy(dst_hbm.at[pl.ds(start, _CB)], dst_v)
        for k in range(4):
            pltpu.sync_copy(ea_hbm.at[pl.ds(k * _E + start, _CB)],
                            ea_v.at[pl.ds(k * _CB, _CB)])
        cp1 = pltpu.async_copy(xl_hbm.at[src_v], xl_v, sem)
        cp2 = pltpu.async_copy(xr_hbm.at[dst_v], xr_v, sem)
        cp1.wait()
        cp2.wait()

        exs = []
        for g in range(_NG):
            eak = [ea_v[pl.ds(k * _CB + g * 16, 16)] for k in range(4)]
            dstg = dst_v[pl.ds(g * 16, 16)]
            alpha = jnp.zeros((16,), jnp.float32)
            for i in range(16):
                r = g * 16 + i
                lane = jnp.full((16,), i, jnp.int32)
                eab = [eak[k].at[lane].get(mode="promise_in_bounds")
                       for k in range(4)]
                acc = jnp.zeros((16,), jnp.float32)
                for db in range(8):
                    sl = pl.ds(db * 16, 16)
                    m = (xl_v[r, sl] + xr_v[r, sl]
                         + eab[0] * we_v[pl.ds(0 * _D + db * 16, 16)]
                         + eab[1] * we_v[pl.ds(1 * _D + db * 16, 16)]
                         + eab[2] * we_v[pl.ds(2 * _D + db * 16, 16)]
                         + eab[3] * we_v[pl.ds(3 * _D + db * 16, 16)])
                    m = jnp.maximum(m, _NEG * m)
                    acc = acc + m * att_v[pl.ds(db * 16, 16)]
                alpha = jnp.where(iota16 == i, jnp.sum(acc), alpha)
            ex = jnp.exp(alpha)
            plsc.addupdate_scatter(den_v, [dstg], ex)
            exs.append(ex)

        @pl.when(j > 0)
        def _():
            pltpu.make_async_copy(w_v, num_sh.at[ds_v], ssem).wait()

        for g in range(_NG):
            ex = exs[g]
            for i in range(16):
                r = g * 16 + i
                lane = jnp.full((16,), i, jnp.int32)
                exb = ex.at[lane].get(mode="promise_in_bounds")
                for db in range(8):
                    sl = pl.ds(db * 16, 16)
                    w_v[r, sl] = exb * xl_v[r, sl]
            ds_v[pl.ds(g * 16, 16)] = dst_v[pl.ds(g * 16, 16)]
        pltpu.async_copy(w_v, num_sh.at[ds_v], ssem, add=True)
        return carry

    lax.fori_loop(0, _NCHUNK, chunk_body, 0)
    pltpu.make_async_copy(w_v, num_sh.at[ds_v], ssem).wait()
    pltpu.sync_copy(den_v, den_out.at[wid])
    plsc.subcore_barrier()
    pltpu.sync_copy(num_sh.at[pl.ds(sid * _RPT, _RPT)],
                    num_out.at[cid, pl.ds(sid * _RPT, _RPT)])


_edge_call = pl.kernel(
    _edge_body,
    out_type=[
        jax.ShapeDtypeStruct((_NCORE, _NPAD, _D), jnp.float32),
        jax.ShapeDtypeStruct((_NW, _NPAD), jnp.float32),
    ],
    mesh=plsc.VectorSubcoreMesh(core_axis_name="c", subcore_axis_name="s"),
    compiler_params=pltpu.CompilerParams(needs_layout_passes=False),
    scratch_types=[
        pltpu.VMEM((_CB,), jnp.int32),
        pltpu.VMEM((_CB,), jnp.int32),
        pltpu.VMEM((_CB,), jnp.int32),
        pltpu.VMEM((4 * _CB,), jnp.float32),
        pltpu.VMEM((_CB, _D), jnp.float32),
        pltpu.VMEM((_CB, _D), jnp.float32),
        pltpu.VMEM((_CB, _D), jnp.float32),
        pltpu.VMEM((_D,), jnp.float32),
        pltpu.VMEM((4 * _D,), jnp.float32),
        pltpu.VMEM((_NPAD,), jnp.float32),
        pltpu.VMEM_SHARED((_NPAD, _D), jnp.float32),
        pltpu.SemaphoreType.DMA,
        pltpu.SemaphoreType.DMA,
    ],
)


def _post_body(num_ref, den_ref, bias_ref, bnw_ref, bnb_ref, out_ref):
    s = num_ref[0] + num_ref[1]
    dn = jnp.sum(den_ref[...], axis=0)
    h = s / (dn + 1e-16) + bias_ref[...]
    h = h / np.float32(np.sqrt(1.0 + _EPS_BN)) * bnw_ref[...] + bnb_ref[...]
    out_ref[...] = jnp.maximum(h, 0.0)


def _post(num, den, bias, bnw, bnb):
    den3 = den.reshape(_NW, _NPAD, 1)
    return pl.pallas_call(
        _post_body,
        grid=(_N // _BLK,),
        in_specs=[
            pl.BlockSpec((_NCORE, _BLK, _D), lambda i: (0, i, 0)),
            pl.BlockSpec((_NW, _BLK, 1), lambda i: (0, i, 0)),
            pl.BlockSpec((_D,), lambda i: (0,)),
            pl.BlockSpec((_D,), lambda i: (0,)),
            pl.BlockSpec((_D,), lambda i: (0,)),
        ],
        out_specs=pl.BlockSpec((_BLK, _D), lambda i: (i, 0)),
        out_shape=jax.ShapeDtypeStruct((_N, _D), jnp.float32),
    )(num, den3, bias, bnw, bnb)


def _final_body(mean_ref, lv_ref, eps_ref, batch_ref, fc1w_ref, fc1b_ref,
                fc2w_ref, fc2b_ref, out_ref, acc, cnt):
    i = pl.program_id(0)
    nb = pl.num_programs(0)

    @pl.when(i == 0)
    def _():
        acc[...] = jnp.zeros_like(acc)
        cnt[...] = jnp.zeros_like(cnt)

    z = mean_ref[...] + eps_ref[...] * jnp.exp(0.5 * lv_ref[...])
    b = batch_ref[0]
    g_iota = jax.lax.broadcasted_iota(jnp.int32, (_G, _BLK), 0)
    onehot = (b == g_iota).astype(jnp.float32)
    acc[...] += jnp.dot(onehot, z, preferred_element_type=jnp.float32)
    cnt[...] += jnp.broadcast_to(jnp.sum(onehot, axis=1, keepdims=True),
                                 (_G, _D))

    @pl.when(i == nb - 1)
    def _():
        pooled = acc[...] / jnp.maximum(cnt[...], 1.0)
        h = jnp.maximum(
            jnp.dot(pooled, fc1w_ref[...], preferred_element_type=jnp.float32)
            + fc1b_ref[...], 0.0)
        logits = jnp.dot(h, fc2w_ref[...],
                         preferred_element_type=jnp.float32) + fc2b_ref[...]
        m = jnp.max(logits, axis=1, keepdims=True)
        lse = m + jnp.log(jnp.sum(jnp.exp(logits - m), axis=1, keepdims=True))
        out_ref[...] = logits - lse


def _final_stage(mean, log_var, eps, batch, fc1_W, fc1_b, fc2_W, fc2_b):
    fc2w_p = jnp.zeros((_D, _D), jnp.float32).at[:, :_NCLS].set(fc2_W)
    fc2b_p = jnp.full((_D,), -1e30, jnp.float32).at[:_NCLS].set(fc2_b)
    batch3 = batch.reshape(_N // _BLK, 1, _BLK)
    out = pl.pallas_call(
        _final_body,
        grid=(_N // _BLK,),
        in_specs=[
            pl.BlockSpec((_BLK, _D), lambda i: (i, 0)),
            pl.BlockSpec((_BLK, _D), lambda i: (i, 0)),
            pl.BlockSpec((_BLK, _D), lambda i: (i, 0)),
            pl.BlockSpec((1, 1, _BLK), lambda i: (i, 0, 0)),
            pl.BlockSpec((_D, _D), lambda i: (0, 0)),
            pl.BlockSpec((_D,), lambda i: (0,)),
            pl.BlockSpec((_D, _D), lambda i: (0, 0)),
            pl.BlockSpec((_D,), lambda i: (0,)),
        ],
        out_specs=pl.BlockSpec((_G, _D), lambda i: (0, 0)),
        out_shape=jax.ShapeDtypeStruct((_G, _D), jnp.float32),
        scratch_shapes=[
            pltpu.VMEM((_G, _D), jnp.float32),
            pltpu.VMEM((_G, _D), jnp.float32),
        ],
    )(mean, log_var, eps, batch3, fc1_W, fc1_b, fc2w_p, fc2b_p)
    return out[:, :_NCLS]


def _gatv2(x, src, dst, ea, z2d, z1d, Wl, bl, Wr, br_, We, att, bias, bnw,
           bnb):
    xl, xr = _mm(x, Wl, bl, Wr, br_)
    ea_t = ea.T.reshape(4 * _E)
    num, den = _edge_call(xl, xr, src, dst, ea_t, att, We.reshape(4 * _D),
                          z2d, z1d)
    return _post(num, den, bias, bnw, bnb)


def kernel(x, edge_index, edge_attr, batch, Wl_m, bl_m, Wr_m, br_m, We_m,
           att_m, bias_m, bnw_m, bnb_m, Wl_v, bl_v, Wr_v, br_v, We_v, att_v,
           bias_v, bnw_v, bnb_v, fc1_W, fc1_b, fc2_W, fc2_b):
    src, dst = edge_index[0], edge_index[1]
    z2d = jnp.zeros((_RPT, _D), jnp.float32)
    z1d = jnp.zeros((_NPAD,), jnp.float32)
    mean = x
    log_var = x
    for i in range(2):
        mean = _gatv2(mean, src, dst, edge_attr, z2d, z1d, Wl_m[i], bl_m[i],
                      Wr_m[i], br_m[i], We_m[i], att_m[i], bias_m[i],
                      bnw_m[i], bnb_m[i])
        log_var = _gatv2(log_var, src, dst, edge_attr, z2d, z1d, Wl_v[i],
                         bl_v[i], Wr_v[i], br_v[i], We_v[i], att_v[i],
                         bias_v[i], bnw_v[i], bnb_v[i])
    eps = jax.random.normal(jax.random.key(42), (_N, _D), jnp.float32)
    logp = _final_stage(mean, log_var, eps, batch, fc1_W, fc1_b, fc2_W, fc2_b)
    return (logp, mean, log_var)


# speedup vs baseline: 1.7208x; 1.7208x over previous
import functools

import numpy as np
import jax
import jax.numpy as jnp
from jax import lax
from jax.experimental import pallas as pl
from jax.experimental.pallas import tpu as pltpu
from jax.experimental.pallas import tpu_sc as plsc

_N = 10000
_E = 320000
_D = 128
_G = 64
_NCLS = 2
_NEG = 0.2
_EPS_BN = 1e-5
_BLK = 1000

_NCORE = 2
_NSUB = 16
_NW = _NCORE * _NSUB
_EPW = _E // _NW
_CB = 80
_NCHUNK = _EPW // _CB
_NG = _CB // 16
_NPAD = 10240
_RPT = _NPAD // _NSUB


def _mm_body(x_ref, wl_ref, bl_ref, wr_ref, br_ref, xl_ref, xr_ref):
    x = x_ref[...]
    xl_ref[...] = jnp.dot(x, wl_ref[...],
                          preferred_element_type=jnp.float32) + bl_ref[...]
    xr_ref[...] = jnp.dot(x, wr_ref[...],
                          preferred_element_type=jnp.float32) + br_ref[...]


def _mm(x, Wl, bl, Wr, br_):
    return pl.pallas_call(
        _mm_body,
        grid=(_N // _BLK,),
        in_specs=[
            pl.BlockSpec((_BLK, _D), lambda i: (i, 0)),
            pl.BlockSpec((_D, _D), lambda i: (0, 0)),
            pl.BlockSpec((_D,), lambda i: (0,)),
            pl.BlockSpec((_D, _D), lambda i: (0, 0)),
            pl.BlockSpec((_D,), lambda i: (0,)),
        ],
        out_specs=[
            pl.BlockSpec((_BLK, _D), lambda i: (i, 0)),
            pl.BlockSpec((_BLK, _D), lambda i: (i, 0)),
        ],
        out_shape=[
            jax.ShapeDtypeStruct((_N, _D), jnp.float32),
            jax.ShapeDtypeStruct((_N, _D), jnp.float32),
        ],
    )(x, Wl, bl, Wr, br_)


def _edge_body(xl_hbm, xr_hbm, src_hbm, dst_hbm, ea_hbm, attb_hbm, web_hbm,
               z2d_hbm, z1d_hbm, num_out, den_out,
               src_v, dst_v, ds_v, ea_v, xl_v, xr_v, w_v, att_v, we_v, den_v,
               num_sh, sem, ssem):
    cid = lax.axis_index("c")
    sid = lax.axis_index("s")
    wid = cid * _NSUB + sid

    pltpu.sync_copy(attb_hbm, att_v)
    pltpu.sync_copy(web_hbm, we_v)
    pltpu.sync_copy(z1d_hbm, den_v)
    pltpu.sync_copy(z2d_hbm, num_sh.at[pl.ds(sid * _RPT, _RPT)])
    plsc.subcore_barrier()

    iota16 = lax.iota(jnp.int32, 16)
    rowvs = [iota16 + 16 * g for g in range(_NG)]

    def chunk_body(j, carry):
        start = pl.multiple_of(wid * _EPW + j * _CB, 8)
        pltpu.sync_copy(src_hbm.at[pl.ds(start, _CB)], src_v)
        pltpu.sync_copy(dst_hbm.at[pl.ds(start, _CB)], dst_v)
        for k in range(4):
            pltpu.sync_copy(ea_hbm.at[pl.ds(k * _E + start, _CB)],
                            ea_v.at[pl.ds(k * _CB, _CB)])
        cp1 = pltpu.async_copy(xl_hbm.at[src_v], xl_v, sem)
        cp2 = pltpu.async_copy(xr_hbm.at[dst_v], xr_v, sem)
        cp1.wait()
        cp2.wait()

        @pl.when(j > 0)
        def _():
            pltpu.make_async_copy(w_v, num_sh.at[ds_v], ssem).wait()

        def group_body(g, carry2):
            eak = [ea_v[pl.ds(k * _CB + g * 16, 16)] for k in range(4)]
            dstg = dst_v[pl.ds(g * 16, 16)]
            alpha = jnp.zeros((16,), jnp.float32)
            for i in range(16):
                r = g * 16 + i
                lane = jnp.full((16,), i, jnp.int32)
                eab = [eak[k].at[lane].get(mode="promise_in_bounds")
                       for k in range(4)]
                acc = jnp.zeros((16,), jnp.float32)
                for db in range(8):
                    sl = pl.ds(db * 16, 16)
                    m = (xl_v[r, sl] + xr_v[r, sl]
                         + eab[0] * we_v[pl.ds(0 * _D + db * 16, 16)]
                         + eab[1] * we_v[pl.ds(1 * _D + db * 16, 16)]
                         + eab[2] * we_v[pl.ds(2 * _D + db * 16, 16)]
                         + eab[3] * we_v[pl.ds(3 * _D + db * 16, 16)])
                    m = jnp.maximum(m, _NEG * m)
                    acc = acc + m * att_v[pl.ds(db * 16, 16)]
                alpha = jnp.where(iota16 == i, jnp.sum(acc), alpha)
            ex = jnp.exp(alpha)
            plsc.addupdate_scatter(den_v, [dstg], ex)
            for i in range(16):
                r = g * 16 + i
                lane = jnp.full((16,), i, jnp.int32)
                exb = ex.at[lane].get(mode="promise_in_bounds")
                for db in range(8):
                    sl = pl.ds(db * 16, 16)
                    w_v[r, sl] = exb * xl_v[r, sl]
            ds_v[pl.ds(g * 16, 16)] = dstg
            return carry2

        lax.fori_loop(0, _NG, group_body, 0)
        pltpu.async_copy(w_v, num_sh.at[ds_v], ssem, add=True)
        return carry

    lax.fori_loop(0, _NCHUNK, chunk_body, 0)
    pltpu.make_async_copy(w_v, num_sh.at[ds_v], ssem).wait()
    pltpu.sync_copy(den_v, den_out.at[wid])
    plsc.subcore_barrier()
    pltpu.sync_copy(num_sh.at[pl.ds(sid * _RPT, _RPT)],
                    num_out.at[cid, pl.ds(sid * _RPT, _RPT)])


_edge_call = pl.kernel(
    _edge_body,
    out_type=[
        jax.ShapeDtypeStruct((_NCORE, _NPAD, _D), jnp.float32),
        jax.ShapeDtypeStruct((_NW, _NPAD), jnp.float32),
    ],
    mesh=plsc.VectorSubcoreMesh(core_axis_name="c", subcore_axis_name="s"),
    compiler_params=pltpu.CompilerParams(needs_layout_passes=False),
    scratch_types=[
        pltpu.VMEM((_CB,), jnp.int32),
        pltpu.VMEM((_CB,), jnp.int32),
        pltpu.VMEM((_CB,), jnp.int32),
        pltpu.VMEM((4 * _CB,), jnp.float32),
        pltpu.VMEM((_CB, _D), jnp.float32),
        pltpu.VMEM((_CB, _D), jnp.float32),
        pltpu.VMEM((_CB, _D), jnp.float32),
        pltpu.VMEM((_D,), jnp.float32),
        pltpu.VMEM((4 * _D,), jnp.float32),
        pltpu.VMEM((_NPAD,), jnp.float32),
        pltpu.VMEM_SHARED((_NPAD, _D), jnp.float32),
        pltpu.SemaphoreType.DMA,
        pltpu.SemaphoreType.DMA,
    ],
)


def _post_body(num_ref, den_ref, bias_ref, bnw_ref, bnb_ref, out_ref):
    s = num_ref[0] + num_ref[1]
    dn = jnp.sum(den_ref[...], axis=0)
    h = s / (dn + 1e-16) + bias_ref[...]
    h = h / np.float32(np.sqrt(1.0 + _EPS_BN)) * bnw_ref[...] + bnb_ref[...]
    out_ref[...] = jnp.maximum(h, 0.0)


def _post(num, den, bias, bnw, bnb):
    den3 = den.reshape(_NW, _NPAD, 1)
    return pl.pallas_call(
        _post_body,
        grid=(_N // _BLK,),
        in_specs=[
            pl.BlockSpec((_NCORE, _BLK, _D), lambda i: (0, i, 0)),
            pl.BlockSpec((_NW, _BLK, 1), lambda i: (0, i, 0)),
            pl.BlockSpec((_D,), lambda i: (0,)),
            pl.BlockSpec((_D,), lambda i: (0,)),
            pl.BlockSpec((_D,), lambda i: (0,)),
        ],
        out_specs=pl.BlockSpec((_BLK, _D), lambda i: (i, 0)),
        out_shape=jax.ShapeDtypeStruct((_N, _D), jnp.float32),
    )(num, den3, bias, bnw, bnb)


def _final_body(mean_ref, lv_ref, eps_ref, batch_ref, fc1w_ref, fc1b_ref,
                fc2w_ref, fc2b_ref, out_ref, acc, cnt):
    i = pl.program_id(0)
    nb = pl.num_programs(0)

    @pl.when(i == 0)
    def _():
        acc[...] = jnp.zeros_like(acc)
        cnt[...] = jnp.zeros_like(cnt)

    z = mean_ref[...] + eps_ref[...] * jnp.exp(0.5 * lv_ref[...])
    b = batch_ref[0]
    g_iota = jax.lax.broadcasted_iota(jnp.int32, (_G, _BLK), 0)
    onehot = (b == g_iota).astype(jnp.float32)
    acc[...] += jnp.dot(onehot, z, preferred_element_type=jnp.float32)
    cnt[...] += jnp.broadcast_to(jnp.sum(onehot, axis=1, keepdims=True),
                                 (_G, _D))

    @pl.when(i == nb - 1)
    def _():
        pooled = acc[...] / jnp.maximum(cnt[...], 1.0)
        h = jnp.maximum(
            jnp.dot(pooled, fc1w_ref[...], preferred_element_type=jnp.float32)
            + fc1b_ref[...], 0.0)
        logits = jnp.dot(h, fc2w_ref[...],
                         preferred_element_type=jnp.float32) + fc2b_ref[...]
        m = jnp.max(logits, axis=1, keepdims=True)
        lse = m + jnp.log(jnp.sum(jnp.exp(logits - m), axis=1, keepdims=True))
        out_ref[...] = logits - lse


def _final_stage(mean, log_var, eps, batch, fc1_W, fc1_b, fc2_W, fc2_b):
    fc2w_p = jnp.zeros((_D, _D), jnp.float32).at[:, :_NCLS].set(fc2_W)
    fc2b_p = jnp.full((_D,), -1e30, jnp.float32).at[:_NCLS].set(fc2_b)
    batch3 = batch.reshape(_N // _BLK, 1, _BLK)
    out = pl.pallas_call(
        _final_body,
        grid=(_N // _BLK,),
        in_specs=[
            pl.BlockSpec((_BLK, _D), lambda i: (i, 0)),
            pl.BlockSpec((_BLK, _D), lambda i: (i, 0)),
            pl.BlockSpec((_BLK, _D), lambda i: (i, 0)),
            pl.BlockSpec((1, 1, _BLK), lambda i: (i, 0, 0)),
            pl.BlockSpec((_D, _D), lambda i: (0, 0)),
            pl.BlockSpec((_D,), lambda i: (0,)),
            pl.BlockSpec((_D, _D), lambda i: (0, 0)),
            pl.BlockSpec((_D,), lambda i: (0,)),
        ],
        out_specs=pl.BlockSpec((_G, _D), lambda i: (0, 0)),
        out_shape=jax.ShapeDtypeStruct((_G, _D), jnp.float32),
        scratch_shapes=[
            pltpu.VMEM((_G, _D), jnp.float32),
            pltpu.VMEM((_G, _D), jnp.float32),
        ],
    )(mean, log_var, eps, batch3, fc1_W, fc1_b, fc2w_p, fc2b_p)
    return out[:, :_NCLS]


def _gatv2(x, src, dst, ea, z2d, z1d, Wl, bl, Wr, br_, We, att, bias, bnw,
           bnb):
    xl, xr = _mm(x, Wl, bl, Wr, br_)
    ea_t = ea.T.reshape(4 * _E)
    num, den = _edge_call(xl, xr, src, dst, ea_t, att, We.reshape(4 * _D),
                          z2d, z1d)
    return _post(num, den, bias, bnw, bnb)


def kernel(x, edge_index, edge_attr, batch, Wl_m, bl_m, Wr_m, br_m, We_m,
           att_m, bias_m, bnw_m, bnb_m, Wl_v, bl_v, Wr_v, br_v, We_v, att_v,
           bias_v, bnw_v, bnb_v, fc1_W, fc1_b, fc2_W, fc2_b):
    src, dst = edge_index[0], edge_index[1]
    z2d = jnp.zeros((_RPT, _D), jnp.float32)
    z1d = jnp.zeros((_NPAD,), jnp.float32)
    mean = x
    log_var = x
    for i in range(2):
        mean = _gatv2(mean, src, dst, edge_attr, z2d, z1d, Wl_m[i], bl_m[i],
                      Wr_m[i], br_m[i], We_m[i], att_m[i], bias_m[i],
                      bnw_m[i], bnb_m[i])
        log_var = _gatv2(log_var, src, dst, edge_attr, z2d, z1d, Wl_v[i],
                         bl_v[i], Wr_v[i], br_v[i], We_v[i], att_v[i],
                         bias_v[i], bnw_v[i], bnb_v[i])
    eps = jax.random.normal(jax.random.key(42), (_N, _D), jnp.float32)
    logp = _final_stage(mean, log_var, eps, batch, fc1_W, fc1_b, fc2_W, fc2_b)
    return (logp, mean, log_var)


# speedup vs baseline: 2.1600x; 1.2552x over previous
import functools

import numpy as np
import jax
import jax.numpy as jnp
from jax import lax
from jax.experimental import pallas as pl
from jax.experimental.pallas import tpu as pltpu
from jax.experimental.pallas import tpu_sc as plsc

_N = 10000
_E = 320000
_D = 128
_G = 64
_NCLS = 2
_NEG = 0.2
_EPS_BN = 1e-5
_BLK = 1000

_NCORE = 2
_NSUB = 16
_NW = _NCORE * _NSUB
_EPW = _E // _NW
_CB = 80
_NCHUNK = _EPW // _CB
_NG = _CB // 16
_NPAD = 10240
_RPT = _NPAD // _NSUB


def _mm_body(x_ref, wl_ref, bl_ref, wr_ref, br_ref, xl_ref, xr_ref):
    x = x_ref[...]
    xl_ref[...] = jnp.dot(x, wl_ref[...],
                          preferred_element_type=jnp.float32) + bl_ref[...]
    xr_ref[...] = jnp.dot(x, wr_ref[...],
                          preferred_element_type=jnp.float32) + br_ref[...]


def _mm(x, Wl, bl, Wr, br_):
    return pl.pallas_call(
        _mm_body,
        grid=(_N // _BLK,),
        in_specs=[
            pl.BlockSpec((_BLK, _D), lambda i: (i, 0)),
            pl.BlockSpec((_D, _D), lambda i: (0, 0)),
            pl.BlockSpec((_D,), lambda i: (0,)),
            pl.BlockSpec((_D, _D), lambda i: (0, 0)),
            pl.BlockSpec((_D,), lambda i: (0,)),
        ],
        out_specs=[
            pl.BlockSpec((_BLK, _D), lambda i: (i, 0)),
            pl.BlockSpec((_BLK, _D), lambda i: (i, 0)),
        ],
        out_shape=[
            jax.ShapeDtypeStruct((_N, _D), jnp.float32),
            jax.ShapeDtypeStruct((_N, _D), jnp.float32),
        ],
    )(x, Wl, bl, Wr, br_)


def _edge_body(xl_hbm, xr_hbm, sd_hbm, ea_hbm, attb_hbm, web_hbm,
               z2d_hbm, z1d_hbm, num_out, den_out,
               sd_v, ds_v, ea_v, xl_v, xr_v, w_v, att_v, we_v, den_v,
               num_sh, sem, ssem):
    src_v = sd_v.at[pl.ds(0, _CB)]
    dst_v = sd_v.at[pl.ds(_CB, _CB)]
    cid = lax.axis_index("c")
    sid = lax.axis_index("s")
    wid = cid * _NSUB + sid

    pltpu.sync_copy(attb_hbm, att_v)
    pltpu.sync_copy(web_hbm, we_v)
    pltpu.sync_copy(z1d_hbm, den_v)
    pltpu.sync_copy(z2d_hbm, num_sh.at[pl.ds(sid * _RPT, _RPT)])
    plsc.subcore_barrier()

    iota16 = lax.iota(jnp.int32, 16)
    rowvs = [iota16 + 16 * g for g in range(_NG)]

    def chunk_body(j, carry):
        cstart = pl.multiple_of((wid * _NCHUNK + j) * 2 * _CB, 8)
        pltpu.sync_copy(sd_hbm.at[pl.ds(cstart, 2 * _CB)], sd_v)
        estart = pl.multiple_of((wid * _NCHUNK + j) * 4 * _CB, 8)
        pltpu.sync_copy(ea_hbm.at[pl.ds(estart, 4 * _CB)], ea_v)
        cp1 = pltpu.async_copy(xl_hbm.at[src_v], xl_v, sem)
        cp2 = pltpu.async_copy(xr_hbm.at[dst_v], xr_v, sem)
        cp1.wait()
        cp2.wait()

        @pl.when(j > 0)
        def _():
            pltpu.make_async_copy(w_v, num_sh.at[ds_v], ssem).wait()

        def group_body(g, carry2):
            eak = [ea_v[pl.ds(k * _CB + g * 16, 16)] for k in range(4)]
            dstg = sd_v[pl.ds(_CB + g * 16, 16)]
            alpha = jnp.zeros((16,), jnp.float32)
            for i in range(16):
                r = g * 16 + i
                lane = jnp.full((16,), i, jnp.int32)
                eab = [eak[k].at[lane].get(mode="promise_in_bounds")
                       for k in range(4)]
                acc = jnp.zeros((16,), jnp.float32)
                for db in range(8):
                    sl = pl.ds(db * 16, 16)
                    m = (xl_v[r, sl] + xr_v[r, sl]
                         + eab[0] * we_v[pl.ds(0 * _D + db * 16, 16)]
                         + eab[1] * we_v[pl.ds(1 * _D + db * 16, 16)]
                         + eab[2] * we_v[pl.ds(2 * _D + db * 16, 16)]
                         + eab[3] * we_v[pl.ds(3 * _D + db * 16, 16)])
                    m = jnp.maximum(m, _NEG * m)
                    acc = acc + m * att_v[pl.ds(db * 16, 16)]
                alpha = jnp.where(iota16 == i, jnp.sum(acc), alpha)
            ex = jnp.exp(alpha)
            plsc.addupdate_scatter(den_v, [dstg], ex)
            for i in range(16):
                r = g * 16 + i
                lane = jnp.full((16,), i, jnp.int32)
                exb = ex.at[lane].get(mode="promise_in_bounds")
                for db in range(8):
                    sl = pl.ds(db * 16, 16)
                    w_v[r, sl] = exb * xl_v[r, sl]
            ds_v[pl.ds(g * 16, 16)] = dstg
            return carry2

        lax.fori_loop(0, _NG, group_body, 0)
        pltpu.async_copy(w_v, num_sh.at[ds_v], ssem, add=True)
        return carry

    lax.fori_loop(0, _NCHUNK, chunk_body, 0)
    pltpu.make_async_copy(w_v, num_sh.at[ds_v], ssem).wait()
    pltpu.sync_copy(den_v, den_out.at[wid])
    plsc.subcore_barrier()
    pltpu.sync_copy(num_sh.at[pl.ds(sid * _RPT, _RPT)],
                    num_out.at[cid, pl.ds(sid * _RPT, _RPT)])


_edge_call = pl.kernel(
    _edge_body,
    out_type=[
        jax.ShapeDtypeStruct((_NCORE, _NPAD, _D), jnp.float32),
        jax.ShapeDtypeStruct((_NW, _NPAD), jnp.float32),
    ],
    mesh=plsc.VectorSubcoreMesh(core_axis_name="c", subcore_axis_name="s"),
    compiler_params=pltpu.CompilerParams(needs_layout_passes=False),
    scratch_types=[
        pltpu.VMEM((2 * _CB,), jnp.int32),
        pltpu.VMEM((_CB,), jnp.int32),
        pltpu.VMEM((4 * _CB,), jnp.float32),
        pltpu.VMEM((_CB, _D), jnp.float32),
        pltpu.VMEM((_CB, _D), jnp.float32),
        pltpu.VMEM((_CB, _D), jnp.float32),
        pltpu.VMEM((_D,), jnp.float32),
        pltpu.VMEM((4 * _D,), jnp.float32),
        pltpu.VMEM((_NPAD,), jnp.float32),
        pltpu.VMEM_SHARED((_NPAD, _D), jnp.float32),
        pltpu.SemaphoreType.DMA,
        pltpu.SemaphoreType.DMA,
    ],
)


def _post_body(num_ref, den_ref, bias_ref, bnw_ref, bnb_ref, out_ref):
    s = num_ref[0] + num_ref[1]
    dn = jnp.sum(den_ref[...], axis=0)
    h = s / (dn + 1e-16) + bias_ref[...]
    h = h / np.float32(np.sqrt(1.0 + _EPS_BN)) * bnw_ref[...] + bnb_ref[...]
    out_ref[...] = jnp.maximum(h, 0.0)


def _post(num, den, bias, bnw, bnb):
    den3 = den.reshape(_NW, _NPAD, 1)
    return pl.pallas_call(
        _post_body,
        grid=(_N // _BLK,),
        in_specs=[
            pl.BlockSpec((_NCORE, _BLK, _D), lambda i: (0, i, 0)),
            pl.BlockSpec((_NW, _BLK, 1), lambda i: (0, i, 0)),
            pl.BlockSpec((_D,), lambda i: (0,)),
            pl.BlockSpec((_D,), lambda i: (0,)),
            pl.BlockSpec((_D,), lambda i: (0,)),
        ],
        out_specs=pl.BlockSpec((_BLK, _D), lambda i: (i, 0)),
        out_shape=jax.ShapeDtypeStruct((_N, _D), jnp.float32),
    )(num, den3, bias, bnw, bnb)


def _final_body(mean_ref, lv_ref, eps_ref, batch_ref, fc1w_ref, fc1b_ref,
                fc2w_ref, fc2b_ref, out_ref, acc, cnt):
    i = pl.program_id(0)
    nb = pl.num_programs(0)

    @pl.when(i == 0)
    def _():
        acc[...] = jnp.zeros_like(acc)
        cnt[...] = jnp.zeros_like(cnt)

    z = mean_ref[...] + eps_ref[...] * jnp.exp(0.5 * lv_ref[...])
    b = batch_ref[0]
    g_iota = jax.lax.broadcasted_iota(jnp.int32, (_G, _BLK), 0)
    onehot = (b == g_iota).astype(jnp.float32)
    acc[...] += jnp.dot(onehot, z, preferred_element_type=jnp.float32)
    cnt[...] += jnp.broadcast_to(jnp.sum(onehot, axis=1, keepdims=True),
                                 (_G, _D))

    @pl.when(i == nb - 1)
    def _():
        pooled = acc[...] / jnp.maximum(cnt[...], 1.0)
        h = jnp.maximum(
            jnp.dot(pooled, fc1w_ref[...], preferred_element_type=jnp.float32)
            + fc1b_ref[...], 0.0)
        logits = jnp.dot(h, fc2w_ref[...],
                         preferred_element_type=jnp.float32) + fc2b_ref[...]
        m = jnp.max(logits, axis=1, keepdims=True)
        lse = m + jnp.log(jnp.sum(jnp.exp(logits - m), axis=1, keepdims=True))
        out_ref[...] = logits - lse


def _final_stage(mean, log_var, eps, batch, fc1_W, fc1_b, fc2_W, fc2_b):
    fc2w_p = jnp.zeros((_D, _D), jnp.float32).at[:, :_NCLS].set(fc2_W)
    fc2b_p = jnp.full((_D,), -1e30, jnp.float32).at[:_NCLS].set(fc2_b)
    batch3 = batch.reshape(_N // _BLK, 1, _BLK)
    out = pl.pallas_call(
        _final_body,
        grid=(_N // _BLK,),
        in_specs=[
            pl.BlockSpec((_BLK, _D), lambda i: (i, 0)),
            pl.BlockSpec((_BLK, _D), lambda i: (i, 0)),
            pl.BlockSpec((_BLK, _D), lambda i: (i, 0)),
            pl.BlockSpec((1, 1, _BLK), lambda i: (i, 0, 0)),
            pl.BlockSpec((_D, _D), lambda i: (0, 0)),
            pl.BlockSpec((_D,), lambda i: (0,)),
            pl.BlockSpec((_D, _D), lambda i: (0, 0)),
            pl.BlockSpec((_D,), lambda i: (0,)),
        ],
        out_specs=pl.BlockSpec((_G, _D), lambda i: (0, 0)),
        out_shape=jax.ShapeDtypeStruct((_G, _D), jnp.float32),
        scratch_shapes=[
            pltpu.VMEM((_G, _D), jnp.float32),
            pltpu.VMEM((_G, _D), jnp.float32),
        ],
    )(mean, log_var, eps, batch3, fc1_W, fc1_b, fc2w_p, fc2b_p)
    return out[:, :_NCLS]


def _gatv2(x, sd, ea_c, z2d, z1d, Wl, bl, Wr, br_, We, att, bias, bnw,
           bnb):
    xl, xr = _mm(x, Wl, bl, Wr, br_)
    num, den = _edge_call(xl, xr, sd, ea_c, att, We.reshape(4 * _D),
                          z2d, z1d)
    return _post(num, den, bias, bnw, bnb)


def kernel(x, edge_index, edge_attr, batch, Wl_m, bl_m, Wr_m, br_m, We_m,
           att_m, bias_m, bnw_m, bnb_m, Wl_v, bl_v, Wr_v, br_v, We_v, att_v,
           bias_v, bnw_v, bnb_v, fc1_W, fc1_b, fc2_W, fc2_b):
    sd = (edge_index.reshape(2, _NW, _NCHUNK, _CB)
          .transpose(1, 2, 0, 3).reshape(2 * _E))
    ea_c = (edge_attr.T.reshape(4, _NW, _NCHUNK, _CB)
            .transpose(1, 2, 0, 3).reshape(4 * _E))
    z2d = jnp.zeros((_RPT, _D), jnp.float32)
    z1d = jnp.zeros((_NPAD,), jnp.float32)
    mean = x
    log_var = x
    for i in range(2):
        mean = _gatv2(mean, sd, ea_c, z2d, z1d, Wl_m[i], bl_m[i],
                      Wr_m[i], br_m[i], We_m[i], att_m[i], bias_m[i],
                      bnw_m[i], bnb_m[i])
        log_var = _gatv2(log_var, sd, ea_c, z2d, z1d, Wl_v[i],
                         bl_v[i], Wr_v[i], br_v[i], We_v[i], att_v[i],
                         bias_v[i], bnw_v[i], bnb_v[i])
    eps = jax.random.normal(jax.random.key(42), (_N, _D), jnp.float32)
    logp = _final_stage(mean, log_var, eps, batch, fc1_W, fc1_b, fc2_W, fc2_b)
    return (logp, mean, log_var)
